# baseline (speedup 1.0000x reference)
import jax
import jax.numpy as jnp
from jax import lax
from jax.experimental import pallas as pl
from jax.experimental.pallas import tpu as pltpu
from jax.experimental.pallas import tpu_sc as plsc

N = 10000
E = 320000
D_FEAT = 128
HID = 128
NUM_CLASSES = 40

NC = 2
NS = 16
NW = NC * NS
WIN = 128
KW = 80
CHS = 16
E_PAD = NW * KW * WIN
NWIN_TOT = E_PAD // WIN
NP = 10240
ROWS_PER_SUB = NP // NS

BR = 1000


def _make_segsum():
  mesh = plsc.VectorSubcoreMesh(core_axis_name="c", subcore_axis_name="s")

  out_type = jax.ShapeDtypeStruct((NC * NP, HID), jnp.float32)
  scratch = [
      pltpu.VMEM((CHS, WIN), jnp.int32),
      pltpu.VMEM((CHS, WIN), jnp.int32),
      pltpu.VMEM((2, WIN, HID), jnp.float32),
      pltpu.VMEM_SHARED((NP, HID), jnp.float32),
      pltpu.SemaphoreType.DMA,
      pltpu.SemaphoreType.DMA,
  ]

  def body(h_hbm, srcw_hbm, dstw_hbm, z_hbm, acc_out,
           si_v, di_v, rows_v, acc_sh, sg0, sg1):
    sg = (sg0, sg1)
    cid = lax.axis_index("c")
    sid = lax.axis_index("s")
    wid = sid * NC + cid
    r0 = sid * ROWS_PER_SUB
    out0 = cid * NP + r0
    base = wid * KW

    pltpu.sync_copy(z_hbm.at[pl.ds(r0, ROWS_PER_SUB)],
                    acc_sh.at[pl.ds(r0, ROWS_PER_SUB)])
    plsc.subcore_barrier()

    def start_gather(k, b):
      pltpu.async_copy(h_hbm.at[si_v.at[k]], rows_v.at[b], sg[b])

    def wait_gather(k, b):
      pltpu.make_async_copy(h_hbm.at[si_v.at[k]], rows_v.at[b],
                            sg[b]).wait()

    @pl.loop(0, KW // CHS)
    def _(c):
      off = base + c * CHS
      pltpu.sync_copy(srcw_hbm.at[pl.ds(off, CHS)], si_v)
      pltpu.sync_copy(dstw_hbm.at[pl.ds(off, CHS)], di_v)
      start_gather(0, 0)
      start_gather(1, 1)
      for w in range(CHS):
        b = w % 2
        wait_gather(w, b)
        pltpu.sync_copy(rows_v.at[b], acc_sh.at[di_v.at[w]], add=True)
        if w < CHS - 2:
          start_gather(w + 2, b)

    plsc.subcore_barrier()
    pltpu.sync_copy(acc_sh.at[pl.ds(r0, ROWS_PER_SUB)],
                    acc_out.at[pl.ds(out0, ROWS_PER_SUB)])

  return pl.kernel(body, out_type=out_type, mesh=mesh, scratch_types=scratch)


def _make_deg():
  mesh = plsc.VectorSubcoreMesh(core_axis_name="c", subcore_axis_name="s")

  out_type = jax.ShapeDtypeStruct((NC * NP, HID), jnp.float32)
  scratch = [
      pltpu.VMEM((KW, WIN), jnp.int32),
      pltpu.VMEM((WIN, HID), jnp.float32),
      pltpu.VMEM_SHARED((NP, HID), jnp.float32),
      pltpu.SemaphoreType.DMA,
  ]
  G = 5

  def body(dstw_hbm, z_hbm, ones_hbm, deg_out, dstw_v, ones_v, acc_sh, sem):
    cid = lax.axis_index("c")
    sid = lax.axis_index("s")
    wid = sid * NC + cid
    r0 = sid * ROWS_PER_SUB
    out0 = cid * NP + r0

    pltpu.sync_copy(z_hbm.at[pl.ds(r0, ROWS_PER_SUB)],
                    acc_sh.at[pl.ds(r0, ROWS_PER_SUB)])
    pltpu.sync_copy(dstw_hbm.at[wid], dstw_v)
    pltpu.sync_copy(ones_hbm, ones_v)
    plsc.subcore_barrier()

    @pl.loop(0, KW, step=G)
    def _(k):
      for b in range(G):
        pltpu.async_copy(ones_v, acc_sh.at[dstw_v.at[k + b]], sem, add=True)
      for b in range(G):
        pltpu.make_async_copy(ones_v, acc_sh.at[dstw_v.at[k + b]],
                              sem).wait()

    plsc.subcore_barrier()
    pltpu.sync_copy(acc_sh.at[pl.ds(r0, ROWS_PER_SUB)],
                    deg_out.at[pl.ds(out0, ROWS_PER_SUB)])

  return pl.kernel(body, out_type=out_type, mesh=mesh, scratch_types=scratch)


_segsum = _make_segsum()
_deg = _make_deg()


def _dot(a, b):
  return jnp.dot(a, b, preferred_element_type=jnp.float32)


def _linear_tc(x, w, b):
  def body(x_ref, w_ref, b_ref, o_ref):
    o_ref[...] = _dot(x_ref[...], w_ref[...]) + b_ref[...]

  return pl.pallas_call(
      body,
      grid=(N // BR,),
      in_specs=[
          pl.BlockSpec((BR, D_FEAT), lambda i: (i, 0)),
          pl.BlockSpec((D_FEAT, HID), lambda i: (0, 0)),
          pl.BlockSpec((1, HID), lambda i: (0, 0)),
      ],
      out_specs=pl.BlockSpec((BR, HID), lambda i: (i, 0)),
      out_shape=jax.ShapeDtypeStruct((N, HID), jnp.float32),
  )(x, w, b.reshape(1, HID))


def _mean_from(a_ref, d_ref):
  agg = a_ref[0] + a_ref[1]
  deg = jnp.maximum(d_ref[0][:, 0:1] + d_ref[1][:, 0:1], 1.0)
  return agg / deg


def _sage_tc(h, acc, deg, ws, wn, b):
  def body(h_ref, a_ref, d_ref, ws_ref, wn_ref, b_ref, o_ref):
    mean = _mean_from(a_ref, d_ref)
    o_ref[...] = (_dot(h_ref[...], ws_ref[...]) + _dot(mean, wn_ref[...])
                  + b_ref[...])

  return pl.pallas_call(
      body,
      grid=(N // BR,),
      in_specs=[
          pl.BlockSpec((BR, HID), lambda i: (i, 0)),
          pl.BlockSpec((NC, BR, HID), lambda i: (0, i, 0)),
          pl.BlockSpec((NC, BR, HID), lambda i: (0, i, 0)),
          pl.BlockSpec((HID, HID), lambda i: (0, 0)),
          pl.BlockSpec((HID, HID), lambda i: (0, 0)),
          pl.BlockSpec((1, HID), lambda i: (0, 0)),
      ],
      out_specs=pl.BlockSpec((BR, HID), lambda i: (i, 0)),
      out_shape=jax.ShapeDtypeStruct((N, HID), jnp.float32),
  )(h, acc, deg, ws, wn, b.reshape(1, HID))


def _sage_mix_tc(res0, res1, acc, deg, ws, wn, b1, wm0, wm1, wm2, bm):
  def body(r0_ref, r1_ref, a_ref, d_ref, ws_ref, wn_ref, b1_ref,
           wm0_ref, wm1_ref, wm2_ref, bm_ref, o_ref):
    mean = _mean_from(a_ref, d_ref)
    res2 = (_dot(r1_ref[...], ws_ref[...]) + _dot(mean, wn_ref[...])
            + b1_ref[...])
    h = (_dot(r0_ref[...], wm0_ref[...]) + _dot(r1_ref[...], wm1_ref[...])
         + _dot(res2, wm2_ref[...]) + bm_ref[...])
    o_ref[...] = jnp.maximum(h, 0.0)

  wspec = pl.BlockSpec((HID, HID), lambda i: (0, 0))
  bspec = pl.BlockSpec((1, HID), lambda i: (0, 0))
  return pl.pallas_call(
      body,
      grid=(N // BR,),
      in_specs=[
          pl.BlockSpec((BR, HID), lambda i: (i, 0)),
          pl.BlockSpec((BR, HID), lambda i: (i, 0)),
          pl.BlockSpec((NC, BR, HID), lambda i: (0, i, 0)),
          pl.BlockSpec((NC, BR, HID), lambda i: (0, i, 0)),
          wspec, wspec, bspec, wspec, wspec, wspec, bspec,
      ],
      out_specs=pl.BlockSpec((BR, HID), lambda i: (i, 0)),
      out_shape=jax.ShapeDtypeStruct((N, HID), jnp.float32),
  )(res0, res1, acc, deg, ws, wn, b1.reshape(1, HID),
    wm0, wm1, wm2, bm.reshape(1, HID))


def _final_tc(res3, acc, deg, ws, wn, b2, wc, bc):
  def body(r3_ref, a_ref, d_ref, ws_ref, wn_ref, b2_ref, wc_ref, bc_ref,
           o_ref):
    mean = _mean_from(a_ref, d_ref)
    res4 = (_dot(r3_ref[...], ws_ref[...]) + _dot(mean, wn_ref[...])
            + b2_ref[...])
    logits = _dot(res4, wc_ref[...]) + bc_ref[...]
    m = jnp.max(logits, axis=1, keepdims=True)
    shifted = logits - m
    lse = jnp.log(jnp.sum(jnp.exp(shifted), axis=1, keepdims=True))
    o_ref[...] = shifted - lse

  return pl.pallas_call(
      body,
      grid=(N // BR,),
      in_specs=[
          pl.BlockSpec((BR, HID), lambda i: (i, 0)),
          pl.BlockSpec((NC, BR, HID), lambda i: (0, i, 0)),
          pl.BlockSpec((NC, BR, HID), lambda i: (0, i, 0)),
          pl.BlockSpec((HID, HID), lambda i: (0, 0)),
          pl.BlockSpec((HID, HID), lambda i: (0, 0)),
          pl.BlockSpec((1, HID), lambda i: (0, 0)),
          pl.BlockSpec((HID, NUM_CLASSES), lambda i: (0, 0)),
          pl.BlockSpec((1, NUM_CLASSES), lambda i: (0, 0)),
      ],
      out_specs=pl.BlockSpec((BR, NUM_CLASSES), lambda i: (i, 0)),
      out_shape=jax.ShapeDtypeStruct((N, NUM_CLASSES), jnp.float32),
  )(res3, acc, deg, ws, wn, b2.reshape(1, HID), wc,
    bc.reshape(1, NUM_CLASSES))


def kernel(x, edge_index, edge_attr, W_lin, b_lin, Ws0, Wn0, bs0,
           Ws1, Wn1, bs1, Ws2, Wn2, bs2, W_mlp, b_mlp, W_cls, b_cls, gate):
  pad = E_PAD - E
  src = jnp.concatenate(
      [edge_index[0], jnp.zeros((pad,), jnp.int32)]).reshape(NW, KW, WIN)
  dst = jnp.concatenate(
      [edge_index[1], jnp.full((pad,), NP - 1, jnp.int32)]).reshape(
          NW, KW, WIN)
  g = jax.nn.sigmoid(gate)
  wm0 = W_mlp * g[0]
  wm1 = W_mlp * g[1]
  wm2 = W_mlp * g[2]
  wc = W_cls * g[3]

  z128 = jnp.zeros((NP, HID), jnp.float32)
  ones128 = jnp.ones((WIN, HID), jnp.float32)

  src2 = src.reshape(NWIN_TOT, WIN)
  dst2 = dst.reshape(NWIN_TOT, WIN)

  res0 = _linear_tc(x, W_lin, b_lin)
  deg = _deg(dst, z128, ones128).reshape(NC, NP, HID)
  acc0 = _segsum(res0, src2, dst2, z128).reshape(NC, NP, HID)
  res1 = _sage_tc(res0, acc0, deg, Ws0, Wn0, bs0)
  acc1 = _segsum(res1, src2, dst2, z128).reshape(NC, NP, HID)
  res3 = _sage_mix_tc(res0, res1, acc1, deg, Ws1, Wn1, bs1, wm0, wm1, wm2,
                      b_mlp)
  acc2 = _segsum(res3, src2, dst2, z128).reshape(NC, NP, HID)
  return _final_tc(res3, acc2, deg, Ws2, Wn2, bs2, wc, b_cls)


# speedup vs baseline: 3.3593x; 3.3593x over previous
import jax
import jax.numpy as jnp
from jax import lax
from jax.experimental import pallas as pl
from jax.experimental.pallas import tpu as pltpu
from jax.experimental.pallas import tpu_sc as plsc

N = 10000
E = 320000
D_FEAT = 128
HID = 128
NUM_CLASSES = 40

NC = 2
NS = 16
NW = NC * NS
WIN = 128
KW = 80
CHS = 16
E_PAD = NW * KW * WIN
NWIN_TOT = E_PAD // WIN
NP = 10240
ROWS_PER_SUB = NP // NS

BR = 1000


def _make_segsum():
  mesh = plsc.VectorSubcoreMesh(core_axis_name="c", subcore_axis_name="s")

  out_type = jax.ShapeDtypeStruct((NC * NP, HID), jnp.float32)
  scratch = [
      pltpu.VMEM((CHS, WIN), jnp.int32),
      pltpu.VMEM((CHS, WIN), jnp.int32),
      pltpu.VMEM((2, WIN, HID), jnp.float32),
      pltpu.VMEM_SHARED((NP, HID), jnp.float32),
      pltpu.SemaphoreType.DMA,
      pltpu.SemaphoreType.DMA,
  ]

  def body(h_hbm, srcw_hbm, dstw_hbm, z_hbm, acc_out,
           si_v, di_v, rows_v, acc_sh, sg0, sg1):
    sg = (sg0, sg1)
    cid = lax.axis_index("c")
    sid = lax.axis_index("s")
    wid = sid * NC + cid
    r0 = sid * ROWS_PER_SUB
    out0 = cid * NP + r0
    base = wid * KW

    pltpu.sync_copy(z_hbm.at[pl.ds(r0, ROWS_PER_SUB)],
                    acc_sh.at[pl.ds(r0, ROWS_PER_SUB)])
    plsc.subcore_barrier()

    def start_gather(k, b):
      pltpu.async_copy(h_hbm.at[si_v.at[k]], rows_v.at[b], sg[b])

    def wait_gather(k, b):
      pltpu.make_async_copy(h_hbm.at[si_v.at[k]], rows_v.at[b],
                            sg[b]).wait()

    @pl.loop(0, KW // CHS)
    def _(c):
      off = base + c * CHS
      pltpu.sync_copy(srcw_hbm.at[pl.ds(off, CHS)], si_v)
      pltpu.sync_copy(dstw_hbm.at[pl.ds(off, CHS)], di_v)
      start_gather(0, 0)
      start_gather(1, 1)
      for w in range(CHS):
        b = w % 2
        wait_gather(w, b)
        pltpu.sync_copy(rows_v.at[b], acc_sh.at[di_v.at[w]], add=True)
        if w < CHS - 2:
          start_gather(w + 2, b)

    plsc.subcore_barrier()
    pltpu.sync_copy(acc_sh.at[pl.ds(r0, ROWS_PER_SUB)],
                    acc_out.at[pl.ds(out0, ROWS_PER_SUB)])

  return pl.kernel(body, out_type=out_type, mesh=mesh, scratch_types=scratch)


def _make_deg():
  mesh = plsc.VectorSubcoreMesh(core_axis_name="c", subcore_axis_name="s")

  out_type = jax.ShapeDtypeStruct((NC * NP, HID), jnp.float32)
  scratch = [
      pltpu.VMEM((KW, WIN), jnp.int32),
      pltpu.VMEM((WIN, HID), jnp.float32),
      pltpu.VMEM_SHARED((NP, HID), jnp.float32),
      pltpu.SemaphoreType.DMA,
  ]
  G = 5

  def body(dstw_hbm, z_hbm, ones_hbm, deg_out, dstw_v, ones_v, acc_sh, sem):
    cid = lax.axis_index("c")
    sid = lax.axis_index("s")
    wid = sid * NC + cid
    r0 = sid * ROWS_PER_SUB
    out0 = cid * NP + r0

    pltpu.sync_copy(z_hbm.at[pl.ds(r0, ROWS_PER_SUB)],
                    acc_sh.at[pl.ds(r0, ROWS_PER_SUB)])
    pltpu.sync_copy(dstw_hbm.at[wid], dstw_v)
    pltpu.sync_copy(ones_hbm, ones_v)
    plsc.subcore_barrier()

    @pl.loop(0, KW, step=G)
    def _(k):
      for b in range(G):
        pltpu.async_copy(ones_v, acc_sh.at[dstw_v.at[k + b]], sem, add=True)
      for b in range(G):
        pltpu.make_async_copy(ones_v, acc_sh.at[dstw_v.at[k + b]],
                              sem).wait()

    plsc.subcore_barrier()
    pltpu.sync_copy(acc_sh.at[pl.ds(r0, ROWS_PER_SUB)],
                    deg_out.at[pl.ds(out0, ROWS_PER_SUB)])

  return pl.kernel(body, out_type=out_type, mesh=mesh, scratch_types=scratch)


_segsum = _make_segsum()
_deg = _make_deg()


def _dot(a, b):
  return jnp.dot(a, b, preferred_element_type=jnp.float32)


def _linear_tc(x, w, b):
  def body(x_ref, w_ref, b_ref, o_ref):
    o_ref[...] = _dot(x_ref[...], w_ref[...]) + b_ref[...]

  return pl.pallas_call(
      body,
      grid=(N // BR,),
      in_specs=[
          pl.BlockSpec((BR, D_FEAT), lambda i: (i, 0)),
          pl.BlockSpec((D_FEAT, HID), lambda i: (0, 0)),
          pl.BlockSpec((1, HID), lambda i: (0, 0)),
      ],
      out_specs=pl.BlockSpec((BR, HID), lambda i: (i, 0)),
      out_shape=jax.ShapeDtypeStruct((N, HID), jnp.float32),
  )(x, w, b.reshape(1, HID))


def _mean_from(a_ref, d_ref):
  agg = a_ref[0] + a_ref[1]
  deg = jnp.maximum(d_ref[0][:, 0:1] + d_ref[1][:, 0:1], 1.0)
  return agg / deg


def _sage_tc(h, acc, deg, ws, wn, b):
  def body(h_ref, a_ref, d_ref, ws_ref, wn_ref, b_ref, o_ref):
    mean = _mean_from(a_ref, d_ref)
    o_ref[...] = (_dot(h_ref[...], ws_ref[...]) + _dot(mean, wn_ref[...])
                  + b_ref[...])

  return pl.pallas_call(
      body,
      grid=(N // BR,),
      in_specs=[
          pl.BlockSpec((BR, HID), lambda i: (i, 0)),
          pl.BlockSpec((NC, BR, HID), lambda i: (0, i, 0)),
          pl.BlockSpec((NC, BR, HID), lambda i: (0, i, 0)),
          pl.BlockSpec((HID, HID), lambda i: (0, 0)),
          pl.BlockSpec((HID, HID), lambda i: (0, 0)),
          pl.BlockSpec((1, HID), lambda i: (0, 0)),
      ],
      out_specs=pl.BlockSpec((BR, HID), lambda i: (i, 0)),
      out_shape=jax.ShapeDtypeStruct((N, HID), jnp.float32),
  )(h, acc, deg, ws, wn, b.reshape(1, HID))


def _sage_mix_tc(res0, res1, acc, deg, ws, wn, b1, wm0, wm1, wm2, bm):
  def body(r0_ref, r1_ref, a_ref, d_ref, ws_ref, wn_ref, b1_ref,
           wm0_ref, wm1_ref, wm2_ref, bm_ref, o_ref):
    mean = _mean_from(a_ref, d_ref)
    res2 = (_dot(r1_ref[...], ws_ref[...]) + _dot(mean, wn_ref[...])
            + b1_ref[...])
    h = (_dot(r0_ref[...], wm0_ref[...]) + _dot(r1_ref[...], wm1_ref[...])
         + _dot(res2, wm2_ref[...]) + bm_ref[...])
    o_ref[...] = jnp.maximum(h, 0.0)

  wspec = pl.BlockSpec((HID, HID), lambda i: (0, 0))
  bspec = pl.BlockSpec((1, HID), lambda i: (0, 0))
  return pl.pallas_call(
      body,
      grid=(N // BR,),
      in_specs=[
          pl.BlockSpec((BR, HID), lambda i: (i, 0)),
          pl.BlockSpec((BR, HID), lambda i: (i, 0)),
          pl.BlockSpec((NC, BR, HID), lambda i: (0, i, 0)),
          pl.BlockSpec((NC, BR, HID), lambda i: (0, i, 0)),
          wspec, wspec, bspec, wspec, wspec, wspec, bspec,
      ],
      out_specs=pl.BlockSpec((BR, HID), lambda i: (i, 0)),
      out_shape=jax.ShapeDtypeStruct((N, HID), jnp.float32),
  )(res0, res1, acc, deg, ws, wn, b1.reshape(1, HID),
    wm0, wm1, wm2, bm.reshape(1, HID))


def _final_tc(res3, acc, deg, ws, wn, b2, wc, bc):
  def body(r3_ref, a_ref, d_ref, ws_ref, wn_ref, b2_ref, wc_ref, bc_ref,
           o_ref):
    mean = _mean_from(a_ref, d_ref)
    res4 = (_dot(r3_ref[...], ws_ref[...]) + _dot(mean, wn_ref[...])
            + b2_ref[...])
    logits = _dot(res4, wc_ref[...]) + bc_ref[...]
    m = jnp.max(logits, axis=1, keepdims=True)
    shifted = logits - m
    lse = jnp.log(jnp.sum(jnp.exp(shifted), axis=1, keepdims=True))
    o_ref[...] = shifted - lse

  return pl.pallas_call(
      body,
      grid=(N // BR,),
      in_specs=[
          pl.BlockSpec((BR, HID), lambda i: (i, 0)),
          pl.BlockSpec((NC, BR, HID), lambda i: (0, i, 0)),
          pl.BlockSpec((NC, BR, HID), lambda i: (0, i, 0)),
          pl.BlockSpec((HID, HID), lambda i: (0, 0)),
          pl.BlockSpec((HID, HID), lambda i: (0, 0)),
          pl.BlockSpec((1, HID), lambda i: (0, 0)),
          pl.BlockSpec((HID, NUM_CLASSES), lambda i: (0, 0)),
          pl.BlockSpec((1, NUM_CLASSES), lambda i: (0, 0)),
      ],
      out_specs=pl.BlockSpec((BR, NUM_CLASSES), lambda i: (i, 0)),
      out_shape=jax.ShapeDtypeStruct((N, NUM_CLASSES), jnp.float32),
  )(res3, acc, deg, ws, wn, b2.reshape(1, HID), wc,
    bc.reshape(1, NUM_CLASSES))


def kernel(x, edge_index, edge_attr, W_lin, b_lin, Ws0, Wn0, bs0,
           Ws1, Wn1, bs1, Ws2, Wn2, bs2, W_mlp, b_mlp, W_cls, b_cls, gate):
  pad = E_PAD - E
  pad_src = (jnp.arange(pad, dtype=jnp.int32) * 13) % N
  pad_dst = N + (jnp.arange(pad, dtype=jnp.int32) % (NP - N))
  src = jnp.concatenate([edge_index[0], pad_src]).reshape(NW, KW, WIN)
  dst = jnp.concatenate([edge_index[1], pad_dst]).reshape(NW, KW, WIN)
  g = jax.nn.sigmoid(gate)
  wm0 = W_mlp * g[0]
  wm1 = W_mlp * g[1]
  wm2 = W_mlp * g[2]
  wc = W_cls * g[3]

  z128 = jnp.zeros((NP, HID), jnp.float32)
  ones128 = jnp.ones((WIN, HID), jnp.float32)

  src2 = src.reshape(NWIN_TOT, WIN)
  dst2 = dst.reshape(NWIN_TOT, WIN)

  res0 = _linear_tc(x, W_lin, b_lin)
  deg = _deg(dst, z128, ones128).reshape(NC, NP, HID)
  acc0 = _segsum(res0, src2, dst2, z128).reshape(NC, NP, HID)
  res1 = _sage_tc(res0, acc0, deg, Ws0, Wn0, bs0)
  acc1 = _segsum(res1, src2, dst2, z128).reshape(NC, NP, HID)
  res3 = _sage_mix_tc(res0, res1, acc1, deg, Ws1, Wn1, bs1, wm0, wm1, wm2,
                      b_mlp)
  acc2 = _segsum(res3, src2, dst2, z128).reshape(NC, NP, HID)
  return _final_tc(res3, acc2, deg, Ws2, Wn2, bs2, wc, b_cls)


# speedup vs baseline: 3.5748x; 1.0641x over previous
import jax
import jax.numpy as jnp
from jax import lax
from jax.experimental import pallas as pl
from jax.experimental.pallas import tpu as pltpu
from jax.experimental.pallas import tpu_sc as plsc

N = 10000
E = 320000
D_FEAT = 128
HID = 128
NUM_CLASSES = 40

NC = 2
NS = 16
NW = NC * NS
WIN = 128
KW = 80
CHW = 8
CH = KW // CHW
E_PAD = NW * KW * WIN
NWIN_TOT = E_PAD // WIN
NP = 10240
ROWS_PER_SUB = NP // NS

BR = 1000


def _make_segsum():
  mesh = plsc.VectorSubcoreMesh(core_axis_name="c", subcore_axis_name="s")

  out_type = jax.ShapeDtypeStruct((NC * NP, HID), jnp.float32)
  scratch = [
      pltpu.VMEM((2, CHW, WIN), jnp.int32),
      pltpu.VMEM((2, CHW, WIN), jnp.int32),
      pltpu.VMEM((2, WIN, HID), jnp.float32),
      pltpu.VMEM_SHARED((NP, HID), jnp.float32),
      pltpu.SemaphoreType.DMA,
      pltpu.SemaphoreType.DMA,
      pltpu.SemaphoreType.DMA,
  ]

  def body(h_hbm, srcw_hbm, dstw_hbm, z_hbm, acc_out,
           si_v, di_v, rows_v, acc_sh, sg0, sg1, sem_i):
    sg = (sg0, sg1)
    cid = lax.axis_index("c")
    sid = lax.axis_index("s")
    wid = sid * NC + cid
    r0 = sid * ROWS_PER_SUB
    out0 = cid * NP + r0
    base = wid * KW

    pltpu.sync_copy(z_hbm.at[pl.ds(r0, ROWS_PER_SUB)],
                    acc_sh.at[pl.ds(r0, ROWS_PER_SUB)])
    plsc.subcore_barrier()

    def start_gather(ib, k, b):
      pltpu.async_copy(h_hbm.at[si_v.at[ib, k]], rows_v.at[b], sg[b])

    def wait_gather(ib, k, b):
      pltpu.make_async_copy(h_hbm.at[si_v.at[ib, k]], rows_v.at[b],
                            sg[b]).wait()

    def start_idx(c, ib):
      off = base + c * CHW
      pltpu.async_copy(srcw_hbm.at[pl.ds(off, CHW)], si_v.at[ib], sem_i)
      pltpu.async_copy(dstw_hbm.at[pl.ds(off, CHW)], di_v.at[ib], sem_i)

    def wait_idx(ib):
      pltpu.make_async_copy(srcw_hbm.at[pl.ds(0, CHW)], si_v.at[ib],
                            sem_i).wait()
      pltpu.make_async_copy(dstw_hbm.at[pl.ds(0, CHW)], di_v.at[ib],
                            sem_i).wait()

    pltpu.sync_copy(srcw_hbm.at[pl.ds(base, CHW)], si_v.at[0])
    pltpu.sync_copy(dstw_hbm.at[pl.ds(base, CHW)], di_v.at[0])
    start_gather(0, 0, 0)
    start_gather(0, 1, 1)
    start_idx(1, 1)

    @pl.loop(0, CH, step=2)
    def _(c0):
      for cb in (0, 1):
        c = c0 + cb
        ob = 1 - cb
        not_last = c != CH - 1
        for w in range(CHW):
          b = w % 2
          wait_gather(cb, w, b)
          pltpu.sync_copy(rows_v.at[b], acc_sh.at[di_v.at[cb, w]],
                          add=True)
          if w == CHW - 2:
            @pl.when(not_last)
            def _():
              wait_idx(ob)
          if w < CHW - 2:
            start_gather(cb, w + 2, b)
          else:
            @pl.when(not_last)
            def _():
              start_gather(ob, w - (CHW - 2), b)
        @pl.when(c < CH - 2)
        def _():
          start_idx(c + 2, cb)

    plsc.subcore_barrier()
    pltpu.sync_copy(acc_sh.at[pl.ds(r0, ROWS_PER_SUB)],
                    acc_out.at[pl.ds(out0, ROWS_PER_SUB)])

  return pl.kernel(body, out_type=out_type, mesh=mesh, scratch_types=scratch)


def _make_deg():
  mesh = plsc.VectorSubcoreMesh(core_axis_name="c", subcore_axis_name="s")

  out_type = jax.ShapeDtypeStruct((NC * NP, HID), jnp.float32)
  scratch = [
      pltpu.VMEM((KW, WIN), jnp.int32),
      pltpu.VMEM((WIN, HID), jnp.float32),
      pltpu.VMEM_SHARED((NP, HID), jnp.float32),
      pltpu.SemaphoreType.DMA,
  ]
  G = 5

  def body(dstw_hbm, z_hbm, ones_hbm, deg_out, dstw_v, ones_v, acc_sh, sem):
    cid = lax.axis_index("c")
    sid = lax.axis_index("s")
    wid = sid * NC + cid
    r0 = sid * ROWS_PER_SUB
    out0 = cid * NP + r0

    pltpu.sync_copy(z_hbm.at[pl.ds(r0, ROWS_PER_SUB)],
                    acc_sh.at[pl.ds(r0, ROWS_PER_SUB)])
    pltpu.sync_copy(dstw_hbm.at[wid], dstw_v)
    pltpu.sync_copy(ones_hbm, ones_v)
    plsc.subcore_barrier()

    @pl.loop(0, KW, step=G)
    def _(k):
      for b in range(G):
        pltpu.async_copy(ones_v, acc_sh.at[dstw_v.at[k + b]], sem, add=True)
      for b in range(G):
        pltpu.make_async_copy(ones_v, acc_sh.at[dstw_v.at[k + b]],
                              sem).wait()

    plsc.subcore_barrier()
    pltpu.sync_copy(acc_sh.at[pl.ds(r0, ROWS_PER_SUB)],
                    deg_out.at[pl.ds(out0, ROWS_PER_SUB)])

  return pl.kernel(body, out_type=out_type, mesh=mesh, scratch_types=scratch)


_segsum = _make_segsum()
_deg = _make_deg()


def _dot(a, b):
  return jnp.dot(a, b, preferred_element_type=jnp.float32)


def _linear_tc(x, w, b):
  def body(x_ref, w_ref, b_ref, o_ref):
    o_ref[...] = _dot(x_ref[...], w_ref[...]) + b_ref[...]

  return pl.pallas_call(
      body,
      grid=(N // BR,),
      in_specs=[
          pl.BlockSpec((BR, D_FEAT), lambda i: (i, 0)),
          pl.BlockSpec((D_FEAT, HID), lambda i: (0, 0)),
          pl.BlockSpec((1, HID), lambda i: (0, 0)),
      ],
      out_specs=pl.BlockSpec((BR, HID), lambda i: (i, 0)),
      out_shape=jax.ShapeDtypeStruct((N, HID), jnp.float32),
  )(x, w, b.reshape(1, HID))


def _mean_from(a_ref, d_ref):
  agg = a_ref[0] + a_ref[1]
  deg = jnp.maximum(d_ref[0][:, 0:1] + d_ref[1][:, 0:1], 1.0)
  return agg / deg


def _sage_tc(h, acc, deg, ws, wn, b):
  def body(h_ref, a_ref, d_ref, ws_ref, wn_ref, b_ref, o_ref):
    mean = _mean_from(a_ref, d_ref)
    o_ref[...] = (_dot(h_ref[...], ws_ref[...]) + _dot(mean, wn_ref[...])
                  + b_ref[...])

  return pl.pallas_call(
      body,
      grid=(N // BR,),
      in_specs=[
          pl.BlockSpec((BR, HID), lambda i: (i, 0)),
          pl.BlockSpec((NC, BR, HID), lambda i: (0, i, 0)),
          pl.BlockSpec((NC, BR, HID), lambda i: (0, i, 0)),
          pl.BlockSpec((HID, HID), lambda i: (0, 0)),
          pl.BlockSpec((HID, HID), lambda i: (0, 0)),
          pl.BlockSpec((1, HID), lambda i: (0, 0)),
      ],
      out_specs=pl.BlockSpec((BR, HID), lambda i: (i, 0)),
      out_shape=jax.ShapeDtypeStruct((N, HID), jnp.float32),
  )(h, acc, deg, ws, wn, b.reshape(1, HID))


def _sage_mix_tc(res0, res1, acc, deg, ws, wn, b1, wm0, wm1, wm2, bm):
  def body(r0_ref, r1_ref, a_ref, d_ref, ws_ref, wn_ref, b1_ref,
           wm0_ref, wm1_ref, wm2_ref, bm_ref, o_ref):
    mean = _mean_from(a_ref, d_ref)
    res2 = (_dot(r1_ref[...], ws_ref[...]) + _dot(mean, wn_ref[...])
            + b1_ref[...])
    h = (_dot(r0_ref[...], wm0_ref[...]) + _dot(r1_ref[...], wm1_ref[...])
         + _dot(res2, wm2_ref[...]) + bm_ref[...])
    o_ref[...] = jnp.maximum(h, 0.0)

  wspec = pl.BlockSpec((HID, HID), lambda i: (0, 0))
  bspec = pl.BlockSpec((1, HID), lambda i: (0, 0))
  return pl.pallas_call(
      body,
      grid=(N // BR,),
      in_specs=[
          pl.BlockSpec((BR, HID), lambda i: (i, 0)),
          pl.BlockSpec((BR, HID), lambda i: (i, 0)),
          pl.BlockSpec((NC, BR, HID), lambda i: (0, i, 0)),
          pl.BlockSpec((NC, BR, HID), lambda i: (0, i, 0)),
          wspec, wspec, bspec, wspec, wspec, wspec, bspec,
      ],
      out_specs=pl.BlockSpec((BR, HID), lambda i: (i, 0)),
      out_shape=jax.ShapeDtypeStruct((N, HID), jnp.float32),
  )(res0, res1, acc, deg, ws, wn, b1.reshape(1, HID),
    wm0, wm1, wm2, bm.reshape(1, HID))


def _final_tc(res3, acc, deg, ws, wn, b2, wc, bc):
  def body(r3_ref, a_ref, d_ref, ws_ref, wn_ref, b2_ref, wc_ref, bc_ref,
           o_ref):
    mean = _mean_from(a_ref, d_ref)
    res4 = (_dot(r3_ref[...], ws_ref[...]) + _dot(mean, wn_ref[...])
            + b2_ref[...])
    logits = _dot(res4, wc_ref[...]) + bc_ref[...]
    m = jnp.max(logits, axis=1, keepdims=True)
    shifted = logits - m
    lse = jnp.log(jnp.sum(jnp.exp(shifted), axis=1, keepdims=True))
    o_ref[...] = shifted - lse

  return pl.pallas_call(
      body,
      grid=(N // BR,),
      in_specs=[
          pl.BlockSpec((BR, HID), lambda i: (i, 0)),
          pl.BlockSpec((NC, BR, HID), lambda i: (0, i, 0)),
          pl.BlockSpec((NC, BR, HID), lambda i: (0, i, 0)),
          pl.BlockSpec((HID, HID), lambda i: (0, 0)),
          pl.BlockSpec((HID, HID), lambda i: (0, 0)),
          pl.BlockSpec((1, HID), lambda i: (0, 0)),
          pl.BlockSpec((HID, NUM_CLASSES), lambda i: (0, 0)),
          pl.BlockSpec((1, NUM_CLASSES), lambda i: (0, 0)),
      ],
      out_specs=pl.BlockSpec((BR, NUM_CLASSES), lambda i: (i, 0)),
      out_shape=jax.ShapeDtypeStruct((N, NUM_CLASSES), jnp.float32),
  )(res3, acc, deg, ws, wn, b2.reshape(1, HID), wc,
    bc.reshape(1, NUM_CLASSES))


def kernel(x, edge_index, edge_attr, W_lin, b_lin, Ws0, Wn0, bs0,
           Ws1, Wn1, bs1, Ws2, Wn2, bs2, W_mlp, b_mlp, W_cls, b_cls, gate):
  pad = E_PAD - E
  pad_src = (jnp.arange(pad, dtype=jnp.int32) * 13) % N
  pad_dst = N + (jnp.arange(pad, dtype=jnp.int32) % (NP - N))
  src = jnp.concatenate([edge_index[0], pad_src]).reshape(NW, KW, WIN)
  dst = jnp.concatenate([edge_index[1], pad_dst]).reshape(NW, KW, WIN)
  g = jax.nn.sigmoid(gate)
  wm0 = W_mlp * g[0]
  wm1 = W_mlp * g[1]
  wm2 = W_mlp * g[2]
  wc = W_cls * g[3]

  z128 = jnp.zeros((NP, HID), jnp.float32)
  ones128 = jnp.ones((WIN, HID), jnp.float32)

  src2 = src.reshape(NWIN_TOT, WIN)
  dst2 = dst.reshape(NWIN_TOT, WIN)

  res0 = _linear_tc(x, W_lin, b_lin)
  deg = _deg(dst, z128, ones128).reshape(NC, NP, HID)
  acc0 = _segsum(res0, src2, dst2, z128).reshape(NC, NP, HID)
  res1 = _sage_tc(res0, acc0, deg, Ws0, Wn0, bs0)
  acc1 = _segsum(res1, src2, dst2, z128).reshape(NC, NP, HID)
  res3 = _sage_mix_tc(res0, res1, acc1, deg, Ws1, Wn1, bs1, wm0, wm1, wm2,
                      b_mlp)
  acc2 = _segsum(res3, src2, dst2, z128).reshape(NC, NP, HID)
  return _final_tc(res3, acc2, deg, Ws2, Wn2, bs2, wc, b_cls)


# speedup vs baseline: 3.5838x; 1.0025x over previous
import jax
import jax.numpy as jnp
from jax import lax
from jax.experimental import pallas as pl
from jax.experimental.pallas import tpu as pltpu
from jax.experimental.pallas import tpu_sc as plsc

N = 10000
E = 320000
D_FEAT = 128
HID = 128
NUM_CLASSES = 40

NC = 2
NS = 16
NW = NC * NS
WIN = 128
KW = 80
CHW = 8
CH = KW // CHW
E_PAD = NW * KW * WIN
NWIN_TOT = E_PAD // WIN
NP = 10240
ROWS_PER_SUB = NP // NS

BR = 1000


def _make_segsum():
  mesh = plsc.VectorSubcoreMesh(core_axis_name="c", subcore_axis_name="s")

  out_type = jax.ShapeDtypeStruct((NC * NP, HID), jnp.float32)
  scratch = [
      pltpu.VMEM((2, CHW, WIN), jnp.int32),
      pltpu.VMEM((2, CHW, WIN), jnp.int32),
      pltpu.VMEM((2, WIN, HID), jnp.float32),
      pltpu.VMEM_SHARED((NP, HID), jnp.float32),
      pltpu.SemaphoreType.DMA,
      pltpu.SemaphoreType.DMA,
      pltpu.SemaphoreType.DMA,
  ]

  def body(h_hbm, srcw_hbm, dstw_hbm, z_hbm, acc_out,
           si_v, di_v, rows_v, acc_sh, sg0, sg1, sem_i):
    sg = (sg0, sg1)
    cid = lax.axis_index("c")
    sid = lax.axis_index("s")
    wid = sid * NC + cid
    r0 = sid * ROWS_PER_SUB
    out0 = cid * NP + r0
    base = wid * KW

    pltpu.sync_copy(z_hbm.at[pl.ds(r0, ROWS_PER_SUB)],
                    acc_sh.at[pl.ds(r0, ROWS_PER_SUB)])
    plsc.subcore_barrier()

    def start_gather(ib, k, b):
      pltpu.async_copy(h_hbm.at[si_v.at[ib, k]], rows_v.at[b], sg[b])

    def wait_gather(ib, k, b):
      pltpu.make_async_copy(h_hbm.at[si_v.at[ib, k]], rows_v.at[b],
                            sg[b]).wait()

    def start_idx(c, ib):
      off = base + c * CHW
      pltpu.async_copy(srcw_hbm.at[pl.ds(off, CHW)], si_v.at[ib], sem_i)
      pltpu.async_copy(dstw_hbm.at[pl.ds(off, CHW)], di_v.at[ib], sem_i)

    def wait_idx(ib):
      pltpu.make_async_copy(srcw_hbm.at[pl.ds(0, CHW)], si_v.at[ib],
                            sem_i).wait()
      pltpu.make_async_copy(dstw_hbm.at[pl.ds(0, CHW)], di_v.at[ib],
                            sem_i).wait()

    pltpu.sync_copy(srcw_hbm.at[pl.ds(base, CHW)], si_v.at[0])
    pltpu.sync_copy(dstw_hbm.at[pl.ds(base, CHW)], di_v.at[0])
    start_gather(0, 0, 0)
    start_gather(0, 1, 1)
    start_idx(1, 1)

    @pl.loop(0, CH, step=2)
    def _(c0):
      for cb in (0, 1):
        c = c0 + cb
        ob = 1 - cb
        not_last = c != CH - 1
        for w in range(CHW):
          b = w % 2
          wait_gather(cb, w, b)
          pltpu.sync_copy(rows_v.at[b], acc_sh.at[di_v.at[cb, w]],
                          add=True)
          if w == CHW - 2:
            @pl.when(not_last)
            def _():
              wait_idx(ob)
          if w < CHW - 2:
            start_gather(cb, w + 2, b)
          else:
            @pl.when(not_last)
            def _():
              start_gather(ob, w - (CHW - 2), b)
        @pl.when(c < CH - 2)
        def _():
          start_idx(c + 2, cb)

    plsc.subcore_barrier()
    pltpu.sync_copy(acc_sh.at[pl.ds(r0, ROWS_PER_SUB)],
                    acc_out.at[pl.ds(out0, ROWS_PER_SUB)])

  return pl.kernel(body, out_type=out_type, mesh=mesh, scratch_types=scratch)


def _make_deg():
  mesh = plsc.VectorSubcoreMesh(core_axis_name="c", subcore_axis_name="s")

  out_type = jax.ShapeDtypeStruct((NC * NP, HID), jnp.float32)
  scratch = [
      pltpu.VMEM((KW, WIN), jnp.int32),
      pltpu.VMEM((WIN, HID), jnp.float32),
      pltpu.VMEM_SHARED((NP, HID), jnp.float32),
      pltpu.SemaphoreType.DMA,
  ]
  G = 10

  def body(dstw_hbm, z_hbm, ones_hbm, deg_out, dstw_v, ones_v, acc_sh, sem):
    cid = lax.axis_index("c")
    sid = lax.axis_index("s")
    wid = sid * NC + cid
    r0 = sid * ROWS_PER_SUB
    out0 = cid * NP + r0

    pltpu.sync_copy(z_hbm.at[pl.ds(r0, ROWS_PER_SUB)],
                    acc_sh.at[pl.ds(r0, ROWS_PER_SUB)])
    pltpu.sync_copy(dstw_hbm.at[wid], dstw_v)
    pltpu.sync_copy(ones_hbm, ones_v)
    plsc.subcore_barrier()

    @pl.loop(0, KW, step=G)
    def _(k):
      for b in range(G):
        pltpu.async_copy(ones_v, acc_sh.at[dstw_v.at[k + b]], sem, add=True)
      for b in range(G):
        pltpu.make_async_copy(ones_v, acc_sh.at[dstw_v.at[k + b]],
                              sem).wait()

    plsc.subcore_barrier()
    pltpu.sync_copy(acc_sh.at[pl.ds(r0, ROWS_PER_SUB)],
                    deg_out.at[pl.ds(out0, ROWS_PER_SUB)])

  return pl.kernel(body, out_type=out_type, mesh=mesh, scratch_types=scratch)


_segsum = _make_segsum()
_deg = _make_deg()


def _dot(a, b):
  return jnp.dot(a, b, preferred_element_type=jnp.float32)


def _linear_tc(x, w, b):
  def body(x_ref, w_ref, b_ref, o_ref):
    o_ref[...] = _dot(x_ref[...], w_ref[...]) + b_ref[...]

  return pl.pallas_call(
      body,
      grid=(N // BR,),
      in_specs=[
          pl.BlockSpec((BR, D_FEAT), lambda i: (i, 0)),
          pl.BlockSpec((D_FEAT, HID), lambda i: (0, 0)),
          pl.BlockSpec((1, HID), lambda i: (0, 0)),
      ],
      out_specs=pl.BlockSpec((BR, HID), lambda i: (i, 0)),
      out_shape=jax.ShapeDtypeStruct((N, HID), jnp.float32),
  )(x, w, b.reshape(1, HID))


def _mean_from(a_ref, d_ref):
  agg = a_ref[0] + a_ref[1]
  deg = jnp.maximum(d_ref[0][:, 0:1] + d_ref[1][:, 0:1], 1.0)
  return agg / deg


def _sage_tc(h, acc, deg, ws, wn, b):
  def body(h_ref, a_ref, d_ref, ws_ref, wn_ref, b_ref, o_ref):
    mean = _mean_from(a_ref, d_ref)
    o_ref[...] = (_dot(h_ref[...], ws_ref[...]) + _dot(mean, wn_ref[...])
                  + b_ref[...])

  return pl.pallas_call(
      body,
      grid=(N // BR,),
      in_specs=[
          pl.BlockSpec((BR, HID), lambda i: (i, 0)),
          pl.BlockSpec((NC, BR, HID), lambda i: (0, i, 0)),
          pl.BlockSpec((NC, BR, HID), lambda i: (0, i, 0)),
          pl.BlockSpec((HID, HID), lambda i: (0, 0)),
          pl.BlockSpec((HID, HID), lambda i: (0, 0)),
          pl.BlockSpec((1, HID), lambda i: (0, 0)),
      ],
      out_specs=pl.BlockSpec((BR, HID), lambda i: (i, 0)),
      out_shape=jax.ShapeDtypeStruct((N, HID), jnp.float32),
  )(h, acc, deg, ws, wn, b.reshape(1, HID))


def _sage_mix_tc(res0, res1, acc, deg, ws, wn, b1, wm0, wm1, wm2, bm):
  def body(r0_ref, r1_ref, a_ref, d_ref, ws_ref, wn_ref, b1_ref,
           wm0_ref, wm1_ref, wm2_ref, bm_ref, o_ref):
    mean = _mean_from(a_ref, d_ref)
    res2 = (_dot(r1_ref[...], ws_ref[...]) + _dot(mean, wn_ref[...])
            + b1_ref[...])
    h = (_dot(r0_ref[...], wm0_ref[...]) + _dot(r1_ref[...], wm1_ref[...])
         + _dot(res2, wm2_ref[...]) + bm_ref[...])
    o_ref[...] = jnp.maximum(h, 0.0)

  wspec = pl.BlockSpec((HID, HID), lambda i: (0, 0))
  bspec = pl.BlockSpec((1, HID), lambda i: (0, 0))
  return pl.pallas_call(
      body,
      grid=(N // BR,),
      in_specs=[
          pl.BlockSpec((BR, HID), lambda i: (i, 0)),
          pl.BlockSpec((BR, HID), lambda i: (i, 0)),
          pl.BlockSpec((NC, BR, HID), lambda i: (0, i, 0)),
          pl.BlockSpec((NC, BR, HID), lambda i: (0, i, 0)),
          wspec, wspec, bspec, wspec, wspec, wspec, bspec,
      ],
      out_specs=pl.BlockSpec((BR, HID), lambda i: (i, 0)),
      out_shape=jax.ShapeDtypeStruct((N, HID), jnp.float32),
  )(res0, res1, acc, deg, ws, wn, b1.reshape(1, HID),
    wm0, wm1, wm2, bm.reshape(1, HID))


def _final_tc(res3, acc, deg, ws, wn, b2, wc, bc):
  def body(r3_ref, a_ref, d_ref, ws_ref, wn_ref, b2_ref, wc_ref, bc_ref,
           o_ref):
    mean = _mean_from(a_ref, d_ref)
    res4 = (_dot(r3_ref[...], ws_ref[...]) + _dot(mean, wn_ref[...])
            + b2_ref[...])
    logits = _dot(res4, wc_ref[...]) + bc_ref[...]
    m = jnp.max(logits, axis=1, keepdims=True)
    shifted = logits - m
    lse = jnp.log(jnp.sum(jnp.exp(shifted), axis=1, keepdims=True))
    o_ref[...] = shifted - lse

  return pl.pallas_call(
      body,
      grid=(N // BR,),
      in_specs=[
          pl.BlockSpec((BR, HID), lambda i: (i, 0)),
          pl.BlockSpec((NC, BR, HID), lambda i: (0, i, 0)),
          pl.BlockSpec((NC, BR, HID), lambda i: (0, i, 0)),
          pl.BlockSpec((HID, HID), lambda i: (0, 0)),
          pl.BlockSpec((HID, HID), lambda i: (0, 0)),
          pl.BlockSpec((1, HID), lambda i: (0, 0)),
          pl.BlockSpec((HID, NUM_CLASSES), lambda i: (0, 0)),
          pl.BlockSpec((1, NUM_CLASSES), lambda i: (0, 0)),
      ],
      out_specs=pl.BlockSpec((BR, NUM_CLASSES), lambda i: (i, 0)),
      out_shape=jax.ShapeDtypeStruct((N, NUM_CLASSES), jnp.float32),
  )(res3, acc, deg, ws, wn, b2.reshape(1, HID), wc,
    bc.reshape(1, NUM_CLASSES))


def kernel(x, edge_index, edge_attr, W_lin, b_lin, Ws0, Wn0, bs0,
           Ws1, Wn1, bs1, Ws2, Wn2, bs2, W_mlp, b_mlp, W_cls, b_cls, gate):
  pad = E_PAD - E
  pad_src = (jnp.arange(pad, dtype=jnp.int32) * 13) % N
  pad_dst = N + (jnp.arange(pad, dtype=jnp.int32) % (NP - N))
  src = jnp.concatenate([edge_index[0], pad_src]).reshape(NW, KW, WIN)
  dst = jnp.concatenate([edge_index[1], pad_dst]).reshape(NW, KW, WIN)
  g = jax.nn.sigmoid(gate)
  wm0 = W_mlp * g[0]
  wm1 = W_mlp * g[1]
  wm2 = W_mlp * g[2]
  wc = W_cls * g[3]

  z128 = jnp.zeros((NP, HID), jnp.float32)
  ones128 = jnp.ones((WIN, HID), jnp.float32)

  src2 = src.reshape(NWIN_TOT, WIN)
  dst2 = dst.reshape(NWIN_TOT, WIN)

  res0 = _linear_tc(x, W_lin, b_lin)
  deg = _deg(dst, z128, ones128).reshape(NC, NP, HID)
  acc0 = _segsum(res0, src2, dst2, z128).reshape(NC, NP, HID)
  res1 = _sage_tc(res0, acc0, deg, Ws0, Wn0, bs0)
  acc1 = _segsum(res1, src2, dst2, z128).reshape(NC, NP, HID)
  res3 = _sage_mix_tc(res0, res1, acc1, deg, Ws1, Wn1, bs1, wm0, wm1, wm2,
                      b_mlp)
  acc2 = _segsum(res3, src2, dst2, z128).reshape(NC, NP, HID)
  return _final_tc(res3, acc2, deg, Ws2, Wn2, bs2, wc, b_cls)
